# baseline (speedup 1.0000x reference)
import functools

import jax
import jax.numpy as jnp
from jax import lax
from jax.experimental import pallas as pl
from jax.experimental.pallas import tpu as pltpu
from jax.experimental.pallas import tpu_sc as plsc

VOCAB = 1000000
SEQ = 200
D = 64
BATCH = 4096
NW = 32
ROWS = BATCH * SEQ
RPW = ROWS // NW
CH = 128
NCH = RPW // CH
NBUF = 4
LOOK = 2


def _impl_body(x_hbm, tok_hbm, pos_hbm, out_hbm, idx_v, pos_v, buf_v, *sems):
    gsem = sems[:NBUF]
    wsem = sems[NBUF:]
    wid = lax.axis_index("s") * 2 + lax.axis_index("c")
    cbase = wid * NCH
    rbase = wid * RPW

    pltpu.sync_copy(x_hbm.at[pl.ds(cbase, NCH), :], idx_v)
    pltpu.sync_copy(pos_hbm, pos_v.at[pl.ds(0, SEQ), :])
    pltpu.sync_copy(pos_hbm, pos_v.at[pl.ds(SEQ, SEQ), :])

    def issue_gather(c, b):
        pltpu.async_copy(tok_hbm.at[idx_v.at[c]], buf_v.at[b], gsem[b])

    def wait_gather(c, b):
        pltpu.make_async_copy(tok_hbm.at[idx_v.at[c]], buf_v.at[b],
                              gsem[b]).wait()

    def issue_write(c, b):
        pltpu.async_copy(buf_v.at[b],
                         out_hbm.at[pl.ds(rbase + c * CH, CH), :], wsem[b])

    def wait_write(c, b):
        pltpu.make_async_copy(buf_v.at[b],
                              out_hbm.at[pl.ds(rbase + c * CH, CH), :],
                              wsem[b]).wait()

    def add_pos(c, b):
        p = lax.rem(c * CH, SEQ)

        def row(r, carry):
            pr = p + r
            for d in range(D // 16):
                sl = pl.ds(d * 16, 16)
                plsc.addupdate(buf_v.at[b, r, sl], pos_v[pr, sl])
            return carry

        lax.fori_loop(0, CH, row, 0, unroll=4)

    for b in range(LOOK):
        issue_gather(b, b)

    def trip(t, carry):
        c0 = t * NBUF
        for b in range(NBUF):
            c = c0 + b
            nb = (b + LOOK) % NBUF
            cn = c + LOOK

            @pl.when(cn < NCH)
            def _():
                @pl.when(c >= LOOK)
                def _():
                    wait_write(c - LOOK, nb)
                issue_gather(cn, nb)

            wait_gather(c, b)
            add_pos(c, b)
            issue_write(c, b)
        return carry

    lax.fori_loop(0, NCH // NBUF, trip, 0)

    for c in range(NCH - LOOK, NCH):
        wait_write(c, c % NBUF)


_impl = functools.partial(
    pl.kernel,
    out_type=jax.ShapeDtypeStruct((ROWS, D), jnp.float32),
    mesh=plsc.VectorSubcoreMesh(core_axis_name="c", subcore_axis_name="s"),
    compiler_params=pltpu.CompilerParams(use_tc_tiling_on_sc=False),
    scratch_types=[
        pltpu.VMEM((NCH, CH), jnp.int32),
        pltpu.VMEM((2 * SEQ, D), jnp.float32),
        pltpu.VMEM((NBUF, CH, D), jnp.float32),
    ] + [pltpu.SemaphoreType.DMA] * (2 * NBUF),
)(_impl_body)


def kernel(x, token_table, pos_table):
    x2 = x.reshape(ROWS // CH, CH)
    out = _impl(x2, token_table, pos_table)
    return out.reshape(BATCH, SEQ, D)


# speedup vs baseline: 1.2230x; 1.2230x over previous
import functools

import jax
import jax.numpy as jnp
from jax import lax
from jax.experimental import pallas as pl
from jax.experimental.pallas import tpu as pltpu
from jax.experimental.pallas import tpu_sc as plsc

VOCAB = 1000000
SEQ = 200
D = 64
BATCH = 4096
NW = 32
BPW = BATCH // NW
NBUF = 4
LOOK = 2
SPLIT = 128


def _impl_body(x_hbm, tok_hbm, pos_hbm, out_hbm, idx_v, pos_v, buf_v, *sems):
    gsem = sems[:NBUF]
    wsem = sems[NBUF:]
    wid = lax.axis_index("s") * 2 + lax.axis_index("c")
    b0 = wid * BPW

    pltpu.sync_copy(x_hbm.at[pl.ds(b0, BPW), :], idx_v)
    pltpu.sync_copy(pos_hbm, pos_v)

    def issue_gather(c, b):
        pltpu.async_copy(tok_hbm.at[idx_v.at[c, pl.ds(0, SPLIT)]],
                         buf_v.at[b, pl.ds(0, SPLIT), :], gsem[b])
        pltpu.async_copy(tok_hbm.at[idx_v.at[c, pl.ds(SPLIT, SEQ - SPLIT)]],
                         buf_v.at[b, pl.ds(SPLIT, SEQ - SPLIT), :], gsem[b])

    def wait_gather(c, b):
        pltpu.make_async_copy(tok_hbm.at[idx_v.at[c, pl.ds(0, SPLIT)]],
                              buf_v.at[b, pl.ds(0, SPLIT), :], gsem[b]).wait()
        pltpu.make_async_copy(tok_hbm.at[idx_v.at[c, pl.ds(SPLIT, SEQ - SPLIT)]],
                              buf_v.at[b, pl.ds(SPLIT, SEQ - SPLIT), :],
                              gsem[b]).wait()

    def issue_write(c, b):
        pltpu.async_copy(buf_v.at[b], out_hbm.at[b0 + c], wsem[b])

    def wait_write(c, b):
        pltpu.make_async_copy(buf_v.at[b], out_hbm.at[b0 + c], wsem[b]).wait()

    def add_pos(b):
        def row(r, carry):
            for d in range(D // 16):
                sl = pl.ds(d * 16, 16)
                plsc.addupdate(buf_v.at[b, r, sl], pos_v[r, sl])
            return carry

        lax.fori_loop(0, SEQ, row, 0, unroll=4)

    for b in range(LOOK):
        issue_gather(b, b)

    def trip(t, carry):
        c0 = t * NBUF
        for b in range(NBUF):
            c = c0 + b
            nb = (b + LOOK) % NBUF
            cn = c + LOOK

            @pl.when(cn < BPW)
            def _():
                @pl.when(c >= LOOK)
                def _():
                    wait_write(c - LOOK, nb)
                issue_gather(cn, nb)

            wait_gather(c, b)
            add_pos(b)
            issue_write(c, b)
        return carry

    lax.fori_loop(0, BPW // NBUF, trip, 0)

    for c in range(BPW - LOOK, BPW):
        wait_write(c, c % NBUF)


_impl = functools.partial(
    pl.kernel,
    out_type=jax.ShapeDtypeStruct((BATCH, SEQ, D), jnp.float32),
    mesh=plsc.VectorSubcoreMesh(core_axis_name="c", subcore_axis_name="s"),
    compiler_params=pltpu.CompilerParams(use_tc_tiling_on_sc=False),
    scratch_types=[
        pltpu.VMEM((BPW, SEQ), jnp.int32),
        pltpu.VMEM((SEQ, D), jnp.float32),
        pltpu.VMEM((NBUF, SEQ, D), jnp.float32),
    ] + [pltpu.SemaphoreType.DMA] * (2 * NBUF),
)(_impl_body)


def kernel(x, token_table, pos_table):
    return _impl(x, token_table, pos_table)


# speedup vs baseline: 1.7454x; 1.4271x over previous
import functools

import jax
import jax.numpy as jnp
from jax import lax
from jax.experimental import pallas as pl
from jax.experimental.pallas import tpu as pltpu
from jax.experimental.pallas import tpu_sc as plsc

VOCAB = 1000000
SEQ = 200
D = 64
BATCH = 4096
NW = 32
BPW = BATCH // NW
NBUF = 4
LOOK = 2
SPLIT = 128


def _impl_body(x_hbm, tok_hbm, pos_hbm, out_hbm, idx_v, pos_v, buf_v, *sems):
    gsem = sems[:NBUF]
    wsem = sems[NBUF:]
    wid = lax.axis_index("s") * 2 + lax.axis_index("c")
    b0 = wid * BPW

    pltpu.sync_copy(x_hbm.at[pl.ds(b0, BPW), :], idx_v)
    pltpu.sync_copy(pos_hbm, pos_v)

    def issue_gather(c, b):
        pltpu.async_copy(tok_hbm.at[idx_v.at[c, pl.ds(0, SPLIT)]],
                         buf_v.at[b, pl.ds(0, SPLIT), :], gsem[b])
        pltpu.async_copy(tok_hbm.at[idx_v.at[c, pl.ds(SPLIT, SEQ - SPLIT)]],
                         buf_v.at[b, pl.ds(SPLIT, SEQ - SPLIT), :], gsem[b])

    def wait_gather(c, b):
        pltpu.make_async_copy(tok_hbm.at[idx_v.at[c, pl.ds(0, SPLIT)]],
                              buf_v.at[b, pl.ds(0, SPLIT), :], gsem[b]).wait()
        pltpu.make_async_copy(tok_hbm.at[idx_v.at[c, pl.ds(SPLIT, SEQ - SPLIT)]],
                              buf_v.at[b, pl.ds(SPLIT, SEQ - SPLIT), :],
                              gsem[b]).wait()

    def issue_write(c, b):
        pltpu.async_copy(buf_v.at[b], out_hbm.at[b0 + c, :, pl.ds(0, D)],
                         wsem[b])

    def wait_write(c, b):
        pltpu.make_async_copy(buf_v.at[b], out_hbm.at[b0 + c, :, pl.ds(0, D)],
                              wsem[b]).wait()

    def add_pos(b):
        def row(r, carry):
            for d in range(D // 16):
                sl = pl.ds(d * 16, 16)
                plsc.addupdate(buf_v.at[b, r, sl], pos_v[r, sl])
            return carry

        lax.fori_loop(0, SEQ, row, 0, unroll=4)

    for b in range(LOOK):
        issue_gather(b, b)

    def trip(t, carry):
        c0 = t * NBUF
        for b in range(NBUF):
            c = c0 + b
            nb = (b + LOOK) % NBUF
            cn = c + LOOK

            @pl.when(cn < BPW)
            def _():
                @pl.when(c >= LOOK)
                def _():
                    wait_write(c - LOOK, nb)
                issue_gather(cn, nb)

            wait_gather(c, b)
            add_pos(b)
            issue_write(c, b)
        return carry

    lax.fori_loop(0, BPW // NBUF, trip, 0)

    for c in range(BPW - LOOK, BPW):
        wait_write(c, c % NBUF)


_impl = functools.partial(
    pl.kernel,
    out_type=jax.ShapeDtypeStruct((BATCH, SEQ, 2 * D), jnp.float32),
    mesh=plsc.VectorSubcoreMesh(core_axis_name="c", subcore_axis_name="s"),
    compiler_params=pltpu.CompilerParams(use_tc_tiling_on_sc=False),
    scratch_types=[
        pltpu.VMEM((BPW, SEQ), jnp.int32),
        pltpu.VMEM((SEQ, D), jnp.float32),
        pltpu.VMEM((NBUF, SEQ, D), jnp.float32),
    ] + [pltpu.SemaphoreType.DMA] * (2 * NBUF),
)(_impl_body)


def kernel(x, token_table, pos_table):
    tokp = jnp.pad(token_table, ((0, 0), (0, D))).reshape(2 * VOCAB, D)
    out = _impl(x * 2, tokp, pos_table)
    return out[:, :, :D]
